# baseline (speedup 1.0000x reference)
import functools

import jax
import jax.numpy as jnp
from jax import lax
from jax.experimental import pallas as pl
from jax.experimental.pallas import tpu as pltpu
from jax.experimental.pallas import tpu_sc as plsc

EMB = 128
HID = 128
OUT = 91
CH = 256


def _sc_gather_pack(table, idx_flat, start, count):
    info = plsc.get_sparse_core_info()
    nw = info.num_cores * info.num_subcores
    b_per_w = count // nw
    assert b_per_w * nw == count
    nbuf = 2
    n_ch = b_per_w // CH
    assert n_ch * CH == b_per_w
    half = CH // 2

    mesh = plsc.VectorSubcoreMesh(core_axis_name="c", subcore_axis_name="s")

    @functools.partial(
        pl.kernel,
        mesh=mesh,
        out_type=jax.ShapeDtypeStruct((count // 2, EMB), jnp.int32),
        scratch_types=[
            pltpu.VMEM((b_per_w,), jnp.int32),
            pltpu.VMEM((nbuf, CH, EMB), jnp.float32),
            pltpu.VMEM((nbuf, half, EMB), jnp.int32),
            pltpu.SemaphoreType.DMA,
            pltpu.SemaphoreType.DMA,
            pltpu.SemaphoreType.DMA,
            pltpu.SemaphoreType.DMA,
        ],
    )
    def k(table_hbm, idx_hbm, out_hbm, idx_v, rows_v, pk_v, g0, g1, w0, w1):
        wid = lax.axis_index("s") * info.num_cores + lax.axis_index("c")
        base = wid * b_per_w
        base2 = wid * (b_per_w // 2)
        pltpu.sync_copy(idx_hbm.at[pl.ds(start + base, b_per_w)], idx_v)
        gsem = (g0, g1)
        wsem = (w0, w1)
        gcp = [None] * n_ch
        wcp = [None] * n_ch

        def start_gather(c):
            gcp[c] = pltpu.async_copy(
                table_hbm.at[idx_v.at[pl.ds(c * CH, CH)]],
                rows_v.at[c % nbuf],
                gsem[c % nbuf],
            )

        def convert(b):
            def body(r, carry):
                for kk in range(EMB // 16):
                    a = rows_v[b, r, pl.ds(kk * 16, 16)]
                    bb = rows_v[b, r + half, pl.ds(kk * 16, 16)]
                    ua = lax.bitcast_convert_type(a, jnp.int32) + jnp.int32(0x8000)
                    ub = lax.bitcast_convert_type(bb, jnp.int32) + jnp.int32(0x8000)
                    word = lax.shift_right_logical(ua, 16) | (
                        ub & jnp.int32(-65536)
                    )
                    pk_v[b, r, pl.ds(kk * 16, 16)] = word
                return carry

            lax.fori_loop(0, half, body, 0)

        for c in range(min(nbuf, n_ch)):
            start_gather(c)
        for c in range(n_ch):
            gcp[c].wait()
            if c >= nbuf:
                wcp[c - nbuf].wait()
            convert(c % nbuf)
            wcp[c] = pltpu.async_copy(
                pk_v.at[c % nbuf],
                out_hbm.at[pl.ds(base2 + c * half, half)],
                wsem[c % nbuf],
            )
            if c + nbuf < n_ch:
                start_gather(c + nbuf)
        for c in range(max(0, n_ch - nbuf), n_ch):
            wcp[c].wait()

    return k(table, idx_flat)


_GRP = 16


def _mlp_body(x_ref, w3_ref, bh_ref, wo_ref, bo_ref, out_ref):
    ctx, pblk, _ = x_ref.shape
    half = CH // 2
    hi_mask = jnp.int32(-65536)

    def unpack(c):
        w = x_ref[c]
        lo = lax.bitcast_convert_type(lax.shift_left(w, 16), jnp.float32)
        hi = lax.bitcast_convert_type(w & hi_mask, jnp.float32)
        lo4 = lo.reshape(pblk // half, half, EMB)
        hi4 = hi.reshape(pblk // half, half, EMB)
        return jnp.concatenate([lo4, hi4], axis=1).reshape(2 * pblk, EMB)

    acc = jnp.dot(unpack(0), w3_ref[0], preferred_element_type=jnp.float32)
    for c in range(1, ctx):
        acc += jnp.dot(unpack(c), w3_ref[c], preferred_element_type=jnp.float32)
    h = jnp.maximum(acc + bh_ref[...], 0.0)
    out_ref[...] = (
        jnp.dot(h, wo_ref[...], preferred_element_type=jnp.float32) + bo_ref[...]
    )


def _mlp_body_acc(prev_ref, x_ref, w3_ref, bh_ref, wo_ref, bo_ref, out_ref):
    del prev_ref
    _mlp_body(x_ref, w3_ref, bh_ref, wo_ref, bo_ref, out_ref)


def _tc_mlp(xp, W3, b_h, W_o, b_o, bq, row_off, prev=None):
    ctx, bq2, _ = xp.shape
    pblk = _GRP * (CH // 2)
    grid = bq2 // pblk
    blk_off = row_off // (2 * pblk)
    in_specs = [
        pl.BlockSpec((ctx, pblk, EMB), lambda i: (0, i, 0)),
        pl.BlockSpec((ctx, EMB, HID), lambda i: (0, 0, 0)),
        pl.BlockSpec((1, HID), lambda i: (0, 0)),
        pl.BlockSpec((HID, OUT), lambda i: (0, 0)),
        pl.BlockSpec((1, OUT), lambda i: (0, 0)),
    ]
    args = [xp, W3, b_h.reshape(1, HID), W_o, b_o.reshape(1, OUT)]
    body = _mlp_body
    kwargs = {}
    if prev is not None:
        in_specs = [pl.BlockSpec(memory_space=pl.ANY)] + in_specs
        args = [prev] + args
        body = _mlp_body_acc
        kwargs["input_output_aliases"] = {0: 0}
    return pl.pallas_call(
        body,
        grid=(grid,),
        in_specs=in_specs,
        out_specs=pl.BlockSpec((2 * pblk, OUT), lambda i: (blk_off + i, 0)),
        out_shape=jax.ShapeDtypeStruct((bq, OUT), jnp.float32),
        **kwargs,
    )(*args)


def kernel(inputs, emb_table, W_h, b_h, W_o, b_o):
    bq, ctx = inputs.shape
    sizes = (12288, 4096)
    pieces = []
    off = 0
    for s in sizes:
        pieces.append(inputs[off : off + s].T.reshape(-1))
        off += s
    idx_flat = jnp.concatenate(pieces)
    w3 = W_h.reshape(ctx, EMB, HID)
    out = None
    off = 0
    for s in sizes:
        packed = _sc_gather_pack(emb_table, idx_flat, off * ctx, s * ctx)
        xp = packed.reshape(ctx, s // 2, EMB)
        out = _tc_mlp(xp, w3, b_h, W_o, b_o, bq, off, prev=out)
        off += s
    return out


# speedup vs baseline: 1.0258x; 1.0258x over previous
import functools

import jax
import jax.numpy as jnp
from jax import lax
from jax.experimental import pallas as pl
from jax.experimental.pallas import tpu as pltpu
from jax.experimental.pallas import tpu_sc as plsc

EMB = 128
HID = 128
OUT = 91
CH = 256


def _sc_gather_pack(table, idx_flat, start, count):
    info = plsc.get_sparse_core_info()
    nw = info.num_cores * info.num_subcores
    b_per_w = count // nw
    assert b_per_w * nw == count
    nbuf = 2
    n_ch = b_per_w // CH
    assert n_ch * CH == b_per_w
    half = CH // 2

    mesh = plsc.VectorSubcoreMesh(core_axis_name="c", subcore_axis_name="s")

    @functools.partial(
        pl.kernel,
        mesh=mesh,
        out_type=jax.ShapeDtypeStruct((count // 2, EMB), jnp.int32),
        scratch_types=[
            pltpu.VMEM((b_per_w,), jnp.int32),
            pltpu.VMEM((nbuf, CH, EMB), jnp.float32),
            pltpu.VMEM((nbuf, half, EMB), jnp.int32),
            pltpu.SemaphoreType.DMA,
            pltpu.SemaphoreType.DMA,
            pltpu.SemaphoreType.DMA,
            pltpu.SemaphoreType.DMA,
        ],
    )
    def k(table_hbm, idx_hbm, out_hbm, idx_v, rows_v, pk_v, g0, g1, w0, w1):
        wid = lax.axis_index("s") * info.num_cores + lax.axis_index("c")
        base = wid * b_per_w
        base2 = wid * (b_per_w // 2)
        pltpu.sync_copy(idx_hbm.at[pl.ds(start + base, b_per_w)], idx_v)
        gsem = (g0, g1)
        wsem = (w0, w1)
        gcp = [None] * n_ch
        wcp = [None] * n_ch

        def start_gather(c):
            gcp[c] = pltpu.async_copy(
                table_hbm.at[idx_v.at[pl.ds(c * CH, CH)]],
                rows_v.at[c % nbuf],
                gsem[c % nbuf],
            )

        def convert(b):
            def body(r, carry):
                for kk in range(EMB // 16):
                    a = rows_v[b, r, pl.ds(kk * 16, 16)]
                    bb = rows_v[b, r + half, pl.ds(kk * 16, 16)]
                    ua = lax.bitcast_convert_type(a, jnp.int32) + jnp.int32(0x8000)
                    ub = lax.bitcast_convert_type(bb, jnp.int32) + jnp.int32(0x8000)
                    word = lax.shift_right_logical(ua, 16) | (
                        ub & jnp.int32(-65536)
                    )
                    pk_v[b, r, pl.ds(kk * 16, 16)] = word
                return carry

            lax.fori_loop(0, half, body, 0)

        for c in range(min(nbuf, n_ch)):
            start_gather(c)
        for c in range(n_ch):
            gcp[c].wait()
            if c >= nbuf:
                wcp[c - nbuf].wait()
            convert(c % nbuf)
            wcp[c] = pltpu.async_copy(
                pk_v.at[c % nbuf],
                out_hbm.at[pl.ds(base2 + c * half, half)],
                wsem[c % nbuf],
            )
            if c + nbuf < n_ch:
                start_gather(c + nbuf)
        for c in range(max(0, n_ch - nbuf), n_ch):
            wcp[c].wait()

    return k(table, idx_flat)


_GRP = 8


def _mlp_body(x_ref, w3_ref, bh_ref, wo_ref, bo_ref, out_ref):
    ctx, pblk, _ = x_ref.shape
    half = CH // 2
    hi_mask = jnp.int32(-65536)

    def unpack(c):
        w = x_ref[c]
        lo = lax.bitcast_convert_type(lax.shift_left(w, 16), jnp.float32)
        hi = lax.bitcast_convert_type(w & hi_mask, jnp.float32)
        lo4 = lo.reshape(pblk // half, half, EMB)
        hi4 = hi.reshape(pblk // half, half, EMB)
        return jnp.concatenate([lo4, hi4], axis=1).reshape(2 * pblk, EMB)

    acc = jnp.dot(unpack(0), w3_ref[0], preferred_element_type=jnp.float32)
    for c in range(1, ctx):
        acc += jnp.dot(unpack(c), w3_ref[c], preferred_element_type=jnp.float32)
    h = jnp.maximum(acc + bh_ref[...], 0.0)
    out_ref[...] = (
        jnp.dot(h, wo_ref[...], preferred_element_type=jnp.float32) + bo_ref[...]
    )


def _mlp_body_acc(prev_ref, x_ref, w3_ref, bh_ref, wo_ref, bo_ref, out_ref):
    del prev_ref
    _mlp_body(x_ref, w3_ref, bh_ref, wo_ref, bo_ref, out_ref)


def _tc_mlp(xp, W3, b_h, W_o, b_o, bq, row_off, prev=None):
    ctx, bq2, _ = xp.shape
    pblk = _GRP * (CH // 2)
    grid = bq2 // pblk
    blk_off = row_off // (2 * pblk)
    in_specs = [
        pl.BlockSpec((ctx, pblk, EMB), lambda i: (0, i, 0)),
        pl.BlockSpec((ctx, EMB, HID), lambda i: (0, 0, 0)),
        pl.BlockSpec((1, HID), lambda i: (0, 0)),
        pl.BlockSpec((HID, OUT), lambda i: (0, 0)),
        pl.BlockSpec((1, OUT), lambda i: (0, 0)),
    ]
    args = [xp, W3, b_h.reshape(1, HID), W_o, b_o.reshape(1, OUT)]
    body = _mlp_body
    kwargs = {}
    if prev is not None:
        in_specs = [pl.BlockSpec(memory_space=pl.ANY)] + in_specs
        args = [prev] + args
        body = _mlp_body_acc
        kwargs["input_output_aliases"] = {0: 0}
    return pl.pallas_call(
        body,
        grid=(grid,),
        in_specs=in_specs,
        out_specs=pl.BlockSpec((2 * pblk, OUT), lambda i: (blk_off + i, 0)),
        out_shape=jax.ShapeDtypeStruct((bq, OUT), jnp.float32),
        **kwargs,
    )(*args)


def kernel(inputs, emb_table, W_h, b_h, W_o, b_o):
    bq, ctx = inputs.shape
    sizes = (12288, 4096)
    pieces = []
    off = 0
    for s in sizes:
        pieces.append(inputs[off : off + s].T.reshape(-1))
        off += s
    idx_flat = jnp.concatenate(pieces)
    w3 = W_h.reshape(ctx, EMB, HID)
    out = None
    off = 0
    for s in sizes:
        packed = _sc_gather_pack(emb_table, idx_flat, off * ctx, s * ctx)
        xp = packed.reshape(ctx, s // 2, EMB)
        out = _tc_mlp(xp, w3, b_h, W_o, b_o, bq, off, prev=out)
        off += s
    return out
